# baseline (speedup 1.0000x reference)
import functools

import jax
import jax.numpy as jnp
from jax import lax
from jax.experimental import pallas as pl
from jax.experimental.pallas import tpu as pltpu
from jax.experimental.pallas import tpu_sc as plsc

D = 64
IDX_W = 128
CH = 256
GRP = 1024


def _tc_transform(table_t, w):
    v = table_t.shape[1]
    bn = 16384

    def mm(t_ref, w_ref, o_ref):
        o_ref[...] = lax.dot_general(
            t_ref[...], w_ref[...],
            (((0,), (1,)), ((), ())),
            preferred_element_type=jnp.float32,
        ).astype(jnp.bfloat16)

    return pl.pallas_call(
        mm,
        grid=(pl.cdiv(v, bn),),
        in_specs=[
            pl.BlockSpec((D, bn), lambda i: (0, i)),
            pl.BlockSpec((D, D), lambda i: (0, 0)),
        ],
        out_specs=pl.BlockSpec((bn, D), lambda i: (i, 0)),
        out_shape=jax.ShapeDtypeStruct((v, D), jnp.bfloat16),
    )(table_t, w)


def _sc_gather_t(g_table, idx2d, hist, batch):
    n_total = idx2d.shape[0] * IDX_W
    info = plsc.get_sparse_core_info()
    nw = info.num_cores * info.num_subcores
    per_w = n_total // nw
    n_chunks = per_w // CH
    k_streams = CH // IDX_W
    grp_rows = GRP // IDX_W
    chunks_per_grp = GRP // CH
    chunks_per_l = batch // CH
    mesh = plsc.VectorSubcoreMesh(core_axis_name="c", subcore_axis_name="s")

    @functools.partial(
        pl.kernel,
        mesh=mesh,
        out_type=jax.ShapeDtypeStruct((hist, D, batch), jnp.float32),
        compiler_params=pltpu.CompilerParams(
            use_tc_tiling_on_sc=False, needs_layout_passes=False),
        scratch_types=[
            pltpu.VMEM((2, grp_rows, IDX_W), jnp.int32),
            pltpu.VMEM((2, CH, D), jnp.bfloat16),
            pltpu.VMEM((2, D, CH + 1), jnp.float32),
            pltpu.SemaphoreType.DMA,
            pltpu.SemaphoreType.DMA,
            pltpu.SemaphoreType.DMA,
            pltpu.SemaphoreType.DMA,
        ],
    )
    def gather_kernel(g_hbm, idx_hbm, out_hbm, idx_v, rows_v, cols_v,
                      gsa, gsb, wsa, wsb):
        wid = lax.axis_index("s") * info.num_cores + lax.axis_index("c")
        base_c = wid * n_chunks
        lanes = jnp.arange(16, dtype=jnp.int32)

        def stage_idx(j):
            @pl.when(j % chunks_per_grp == 0)
            def _():
                g = (base_c + j) // chunks_per_grp
                row0 = pl.multiple_of(g * grp_rows, grp_rows)
                pltpu.sync_copy(idx_hbm.at[pl.ds(row0, grp_rows)],
                                idx_v.at[g % 2])
            g = (base_c + j) // chunks_per_grp
            return g % 2

        def gather_copies(j, sem):
            slot = ((base_c + j) // chunks_per_grp) % 2
            q = (j % chunks_per_grp) * k_streams
            return [
                pltpu.make_async_copy(
                    g_hbm.at[idx_v.at[slot, q + s]],
                    rows_v.at[j % 2].at[pl.ds(s * IDX_W, IDX_W)],
                    sem,
                )
                for s in range(k_streams)
            ]

        def wb_copy(j, sem):
            c = base_c + j
            l = c // chunks_per_l
            b0 = pl.multiple_of((c % chunks_per_l) * CH, CH)
            return pltpu.make_async_copy(
                cols_v.at[j % 2, :, pl.ds(0, CH)],
                out_hbm.at[l, :, pl.ds(b0, CH)], sem)

        def transpose(j):
            p = j % 2
            pv = jnp.full((16,), p, dtype=jnp.int32)
            lanes2 = lanes * 2

            def trans_b(b8, carry2):
                for bu in range(8):
                    b = b8 * 8 + bu
                    bv = jnp.full((16,), b, dtype=jnp.int32)
                    for k in range(D // 32):
                        vec = rows_v[p, b, pl.ds(k * 32, 32)]
                        ev, od = plsc.unpack(
                            vec, format=plsc.PackFormat.INTERLEAVED)
                        plsc.store_scatter(
                            cols_v, [pv, lanes2 + (k * 32), bv], ev)
                        plsc.store_scatter(
                            cols_v, [pv, lanes2 + (k * 32 + 1), bv], od)
                return carry2

            lax.fori_loop(0, CH // 8, trans_b, 0)

        stage_idx(0)
        for c_ in gather_copies(0, gsa):
            c_.start()

        def body(jj, carry):
            for par, gsem_n, gsem_p, wsem in (
                    (0, gsb, gsa, wsa), (1, gsa, gsb, wsb)):
                j = jj * 2 + par
                jn = j + 1
                @pl.when(jn < n_chunks)
                def _():
                    stage_idx(jn)
                    for c_ in gather_copies(jn, gsem_n):
                        c_.start()
                for c_ in gather_copies(j, gsem_p):
                    c_.wait()
                @pl.when(j >= 2)
                def _():
                    wb_copy(j - 2, wsem).wait()
                transpose(j)
                wb_copy(j, wsem).start()
            return carry

        lax.fori_loop(0, n_chunks // 2, body, 0)
        wb_copy(n_chunks - 2, wsa).wait()
        wb_copy(n_chunks - 1, wsb).wait()

    return gather_kernel(g_table, idx2d)


def kernel(x, emb_table, fc_w):
    batch, hist = x.shape
    g_table = _tc_transform(jnp.swapaxes(emb_table, 0, 1), fc_w)
    idx2d = jnp.swapaxes(x, 0, 1).reshape(-1, IDX_W).astype(jnp.int32)
    out3 = _sc_gather_t(g_table, idx2d, hist, batch)
    return jnp.transpose(out3, (2, 0, 1))


# speedup vs baseline: 1.1387x; 1.1387x over previous
import functools

import jax
import jax.numpy as jnp
from jax import lax
from jax.experimental import pallas as pl
from jax.experimental.pallas import tpu as pltpu
from jax.experimental.pallas import tpu_sc as plsc

D = 64
IDX_W = 128
CH = 256
GRP = 1024


def _tc_transform(table_t, w):
    v = table_t.shape[1]
    bn = 16384

    def mm(t_ref, we_ref, wo_ref, o_ref):
        def half(w_ref):
            res = lax.dot_general(
                t_ref[...], w_ref[...],
                (((0,), (0,)), ((), ())),
                preferred_element_type=jnp.float32,
            ).astype(jnp.bfloat16)
            return lax.bitcast_convert_type(res, jnp.uint16).astype(
                jnp.uint32)
        packed = half(we_ref) | (half(wo_ref) << 16)
        o_ref[...] = lax.bitcast_convert_type(packed, jnp.int32)

    return pl.pallas_call(
        mm,
        grid=(pl.cdiv(v, bn),),
        in_specs=[
            pl.BlockSpec((D, bn), lambda i: (0, i)),
            pl.BlockSpec((D, D // 2), lambda i: (0, 0)),
            pl.BlockSpec((D, D // 2), lambda i: (0, 0)),
        ],
        out_specs=pl.BlockSpec((bn, D // 2), lambda i: (i, 0)),
        out_shape=jax.ShapeDtypeStruct((v, D // 2), jnp.int32),
    )(table_t, jnp.swapaxes(w, 0, 1)[:, 0::2], jnp.swapaxes(w, 0, 1)[:, 1::2])


def _sc_gather_t(g_table, idx2d, hist, batch):
    n_total = idx2d.shape[0] * IDX_W
    info = plsc.get_sparse_core_info()
    nw = info.num_cores * info.num_subcores
    per_w = n_total // nw
    n_chunks = per_w // CH
    k_streams = CH // IDX_W
    grp_rows = GRP // IDX_W
    chunks_per_grp = GRP // CH
    chunks_per_l = batch // CH
    mesh = plsc.VectorSubcoreMesh(core_axis_name="c", subcore_axis_name="s")

    @functools.partial(
        pl.kernel,
        mesh=mesh,
        out_type=jax.ShapeDtypeStruct((hist, D, batch), jnp.float32),
        compiler_params=pltpu.CompilerParams(
            use_tc_tiling_on_sc=False, needs_layout_passes=False),
        scratch_types=[
            pltpu.VMEM((2, grp_rows, IDX_W), jnp.int32),
            pltpu.VMEM((2, CH, D // 2), jnp.int32),
            pltpu.VMEM((2, D, CH + 1), jnp.float32),
            pltpu.SemaphoreType.DMA,
            pltpu.SemaphoreType.DMA,
            pltpu.SemaphoreType.DMA,
            pltpu.SemaphoreType.DMA,
        ],
    )
    def gather_kernel(g_hbm, idx_hbm, out_hbm, idx_v, rows_v, cols_v,
                      gsa, gsb, wsa, wsb):
        wid = lax.axis_index("s") * info.num_cores + lax.axis_index("c")
        base_c = wid * n_chunks
        lanes = jnp.arange(16, dtype=jnp.int32)

        def stage_idx(j):
            @pl.when(j % chunks_per_grp == 0)
            def _():
                g = (base_c + j) // chunks_per_grp
                row0 = pl.multiple_of(g * grp_rows, grp_rows)
                pltpu.sync_copy(idx_hbm.at[pl.ds(row0, grp_rows)],
                                idx_v.at[g % 2])
            g = (base_c + j) // chunks_per_grp
            return g % 2

        def gather_copies(j, sem):
            slot = ((base_c + j) // chunks_per_grp) % 2
            q = (j % chunks_per_grp) * k_streams
            return [
                pltpu.make_async_copy(
                    g_hbm.at[idx_v.at[slot, q + s]],
                    rows_v.at[j % 2].at[pl.ds(s * IDX_W, IDX_W)],
                    sem,
                )
                for s in range(k_streams)
            ]

        def wb_copy(j, sem):
            c = base_c + j
            l = c // chunks_per_l
            b0 = pl.multiple_of((c % chunks_per_l) * CH, CH)
            return pltpu.make_async_copy(
                cols_v.at[j % 2, :, pl.ds(0, CH)],
                out_hbm.at[l, :, pl.ds(b0, CH)], sem)

        def transpose(j):
            p = j % 2
            pv = jnp.full((16,), p, dtype=jnp.int32)
            lanes2 = lanes * 2

            def trans_b(b8, carry2):
                for bu in range(8):
                    b = b8 * 8 + bu
                    bv = jnp.full((16,), b, dtype=jnp.int32)
                    for k in range(D // 32):
                        vec = plsc.bitcast(
                            rows_v[p, b, pl.ds(k * 16, 16)], jnp.bfloat16)
                        ev, od = plsc.unpack(
                            vec, format=plsc.PackFormat.INTERLEAVED)
                        plsc.store_scatter(
                            cols_v, [pv, lanes2 + (k * 32), bv], ev)
                        plsc.store_scatter(
                            cols_v, [pv, lanes2 + (k * 32 + 1), bv], od)
                return carry2

            lax.fori_loop(0, CH // 8, trans_b, 0)

        stage_idx(0)
        for c_ in gather_copies(0, gsa):
            c_.start()

        def body(jj, carry):
            for par, gsem_n, gsem_p, wsem in (
                    (0, gsb, gsa, wsa), (1, gsa, gsb, wsb)):
                j = jj * 2 + par
                jn = j + 1
                @pl.when(jn < n_chunks)
                def _():
                    stage_idx(jn)
                    for c_ in gather_copies(jn, gsem_n):
                        c_.start()
                for c_ in gather_copies(j, gsem_p):
                    c_.wait()
                @pl.when(j >= 2)
                def _():
                    wb_copy(j - 2, wsem).wait()
                transpose(j)
                wb_copy(j, wsem).start()
            return carry

        lax.fori_loop(0, n_chunks // 2, body, 0)
        wb_copy(n_chunks - 2, wsa).wait()
        wb_copy(n_chunks - 1, wsb).wait()

    return gather_kernel(g_table, idx2d)


def kernel(x, emb_table, fc_w):
    batch, hist = x.shape
    g_table = _tc_transform(jnp.swapaxes(emb_table, 0, 1), fc_w)
    idx2d = jnp.swapaxes(x, 0, 1).reshape(-1, IDX_W).astype(jnp.int32)
    out3 = _sc_gather_t(g_table, idx2d, hist, batch)
    return jnp.transpose(out3, (2, 0, 1))
